# Initial kernel scaffold; baseline (speedup 1.0000x reference)
import jax
import jax.numpy as jnp
from jax.experimental import pallas as pl


def kernel(x):
    raise NotImplementedError("write your pallas kernel here")


# speedup vs baseline: 2.0445x; 2.0445x over previous
import jax
import jax.numpy as jnp
from jax.experimental import pallas as pl

_DEPTH = 1000
_B, _T = 1024, 200
_BB = 8


def _onehot_body(x_ref, o_ref):
    x = x_ref[...] % _DEPTH
    d = jax.lax.broadcasted_iota(jnp.int32, (_BB, _T, _DEPTH), 2)
    o_ref[...] = (d == x[:, :, None]).astype(jnp.float32)


def kernel(x):
    return pl.pallas_call(
        _onehot_body,
        grid=(_B // _BB,),
        in_specs=[pl.BlockSpec((_BB, _T), lambda i: (i, 0))],
        out_specs=pl.BlockSpec((_BB, _T, _DEPTH), lambda i: (i, 0, 0)),
        out_shape=jax.ShapeDtypeStruct((_B, _T, _DEPTH), jnp.float32),
    )(x)
